# baseline (device time: 40192 ns/iter reference)
import jax
import jax.numpy as jnp
from jax import lax
from jax.experimental import pallas as pl
from jax.experimental.pallas import tpu as pltpu

N_DEV = 4
B, SQ, SKV_LOC, HQ_LOC, DH = 2, 128, 128, 4, 64
D_MODEL = 512


def kernel(x, Wq, K_ext, V_ext, Wo):
    def body(x_ref, wq_ref, k_ref, v_ref, wo_ref, out_ref,
             send_buf, kv_buf, acc_buf,
             a2a_send_sems, a2a_recv_sems, ring_send_sems, ring_recv_sems):
        my = lax.axis_index("i")
        right = (my + 1) % N_DEV

        barrier = pltpu.get_barrier_semaphore()
        for d in range(1, N_DEV):
            pl.semaphore_signal(
                barrier, inc=1,
                device_id=((my + d) % N_DEV,),
                device_id_type=pl.DeviceIdType.MESH,
            )
        pl.semaphore_wait(barrier, N_DEV - 1)

        for j in range(N_DEV):
            send_buf[j, 0] = k_ref[:, :, 4 * j:4 * j + 4, :].astype(jnp.bfloat16)
            send_buf[j, 1] = v_ref[:, :, 4 * j:4 * j + 4, :].astype(jnp.bfloat16)

        kv_buf[pl.ds(my, 1)] = send_buf[pl.ds(my, 1)]

        sends = []
        for d in range(1, N_DEV):
            tgt = (my + d) % N_DEV
            rdma = pltpu.make_async_remote_copy(
                src_ref=send_buf.at[tgt],
                dst_ref=kv_buf.at[my],
                send_sem=a2a_send_sems.at[tgt],
                recv_sem=a2a_recv_sems.at[my],
                device_id=(tgt,),
                device_id_type=pl.DeviceIdType.MESH,
            )
            rdma.start()
            sends.append(rdma)

        x_b = x_ref[...].astype(jnp.bfloat16)
        wq_b = wq_ref[...].astype(jnp.bfloat16)
        q = lax.dot_general(
            x_b, wq_b, (((2,), (0,)), ((), ())),
            preferred_element_type=jnp.float32,
        ).astype(jnp.bfloat16)

        qi = lax.broadcasted_iota(jnp.int32, (SQ, N_DEV * SKV_LOC), 0) // 64
        kj = lax.broadcasted_iota(jnp.int32, (SQ, N_DEV * SKV_LOC), 1) // 64
        mask = (qi == kj) | ((kj % 4) == (qi % 4))

        for d in range(1, N_DEV):
            src = (my + d) % N_DEV
            recv = pltpu.make_async_remote_copy(
                src_ref=send_buf.at[0],
                dst_ref=kv_buf.at[src],
                send_sem=a2a_send_sems.at[0],
                recv_sem=a2a_recv_sems.at[src],
                device_id=(src,),
                device_id_type=pl.DeviceIdType.MESH,
            )
            recv.wait_recv()
        for rdma in sends:
            rdma.wait_send()

        ctx_heads = []
        for h in range(HQ_LOC):
            q_h = q[:, :, 64 * h:64 * h + 64]
            s_chunks = []
            for j in range(N_DEV):
                k_jh = kv_buf[j, 0][:, :, h, :]
                s_chunks.append(lax.dot_general(
                    q_h, k_jh, (((2,), (2,)), ((0,), (0,))),
                    preferred_element_type=jnp.float32,
                ))
            s = jnp.concatenate(s_chunks, axis=2) * 0.125
            s = jnp.where(mask[None], s, -1e9)
            m = jnp.max(s, axis=-1, keepdims=True)
            w = jnp.exp(s - m)
            w = (w / jnp.sum(w, axis=-1, keepdims=True)).astype(jnp.bfloat16)
            c_h = jnp.zeros((B, SQ, DH), jnp.float32)
            for j in range(N_DEV):
                v_jh = kv_buf[j, 1][:, :, h, :]
                c_h = c_h + lax.dot_general(
                    w[:, :, 128 * j:128 * j + 128], v_jh,
                    (((2,), (1,)), ((0,), (0,))),
                    preferred_element_type=jnp.float32,
                )
            ctx_heads.append(c_h)
        ctx = jnp.concatenate(ctx_heads, axis=2).astype(jnp.bfloat16)

        partial = lax.dot_general(
            ctx, wo_ref[...].astype(jnp.bfloat16), (((2,), (0,)), ((), ())),
            preferred_element_type=jnp.float32,
        )

        acc_buf[pl.ds(my, 1)] = partial.astype(jnp.bfloat16)[None]
        for h in range(N_DEV - 1):
            send_slot = (my - h) % N_DEV
            rdma = pltpu.make_async_remote_copy(
                src_ref=acc_buf.at[send_slot],
                dst_ref=acc_buf.at[send_slot],
                send_sem=ring_send_sems.at[h],
                recv_sem=ring_recv_sems.at[h],
                device_id=(right,),
                device_id_type=pl.DeviceIdType.MESH,
            )
            rdma.start()
            rdma.wait()

        out_ref[...] = jnp.sum(acc_buf[...].astype(jnp.float32), axis=0)

    return pl.pallas_call(
        body,
        out_shape=jax.ShapeDtypeStruct((B, SQ, D_MODEL), jnp.float32),
        in_specs=[pl.BlockSpec(memory_space=pltpu.VMEM)] * 5,
        out_specs=pl.BlockSpec(memory_space=pltpu.VMEM),
        scratch_shapes=[
            pltpu.VMEM((N_DEV, 2, B, SKV_LOC, HQ_LOC, DH), jnp.bfloat16),
            pltpu.VMEM((N_DEV, 2, B, SKV_LOC, HQ_LOC, DH), jnp.bfloat16),
            pltpu.VMEM((N_DEV, B, SQ, D_MODEL), jnp.bfloat16),
            pltpu.SemaphoreType.DMA((N_DEV,)),
            pltpu.SemaphoreType.DMA((N_DEV,)),
            pltpu.SemaphoreType.DMA((N_DEV - 1,)),
            pltpu.SemaphoreType.DMA((N_DEV - 1,)),
        ],
        compiler_params=pltpu.CompilerParams(collective_id=0),
    )(x, Wq, K_ext, V_ext, Wo)


# device time: 33083 ns/iter; 1.2149x vs baseline; 1.2149x over previous
import jax
import jax.numpy as jnp
from jax import lax
from jax.experimental import pallas as pl
from jax.experimental.pallas import tpu as pltpu

N_DEV = 4
B, SQ, SKV_LOC, HQ_LOC, DH = 2, 128, 128, 4, 64
D_MODEL = 512
SENDERS = (0, 2)
SKV_KEPT = len(SENDERS) * SKV_LOC


def kernel(x, Wq, K_ext, V_ext, Wo):
    def body(x_ref, wq_ref, k_ref, v_ref, wo_ref, out_ref,
             send_buf, kv_buf, acc_buf,
             a2a_send_sems, a2a_recv_sems, bc_send_sems, bc_recv_sems):
        my = lax.axis_index("i")

        barrier = pltpu.get_barrier_semaphore()
        for d in range(1, N_DEV):
            pl.semaphore_signal(
                barrier, inc=1,
                device_id=((my + d) % N_DEV,),
                device_id_type=pl.DeviceIdType.MESH,
            )
        pl.semaphore_wait(barrier, N_DEV - 1)

        for slot, s in enumerate(SENDERS):
            @pl.when(my == s)
            def _(slot=slot, s=s):
                kv_buf[slot, 0] = k_ref[:, :, 4 * s:4 * s + 4, :].astype(jnp.bfloat16)
                kv_buf[slot, 1] = v_ref[:, :, 4 * s:4 * s + 4, :].astype(jnp.bfloat16)
                for j in range(N_DEV):
                    if j == s:
                        continue
                    send_buf[j, 0] = k_ref[:, :, 4 * j:4 * j + 4, :].astype(jnp.bfloat16)
                    send_buf[j, 1] = v_ref[:, :, 4 * j:4 * j + 4, :].astype(jnp.bfloat16)
                    rdma = pltpu.make_async_remote_copy(
                        src_ref=send_buf.at[j],
                        dst_ref=kv_buf.at[slot],
                        send_sem=a2a_send_sems.at[j],
                        recv_sem=a2a_recv_sems.at[slot],
                        device_id=(j,),
                        device_id_type=pl.DeviceIdType.MESH,
                    )
                    rdma.start()

        x_b = x_ref[...].astype(jnp.bfloat16)
        wq_b = wq_ref[...].astype(jnp.bfloat16)
        q = lax.dot_general(
            x_b, wq_b, (((2,), (0,)), ((), ())),
            preferred_element_type=jnp.float32,
        ).astype(jnp.bfloat16)

        qi = lax.broadcasted_iota(jnp.int32, (SQ, SKV_KEPT), 0) // 64
        kc = (lax.broadcasted_iota(jnp.int32, (SQ, SKV_KEPT), 1) // 64) % 2
        mask = qi == kc

        for slot, s in enumerate(SENDERS):
            @pl.when(my != s)
            def _(slot=slot, s=s):
                recv = pltpu.make_async_remote_copy(
                    src_ref=send_buf.at[0],
                    dst_ref=kv_buf.at[slot],
                    send_sem=a2a_send_sems.at[0],
                    recv_sem=a2a_recv_sems.at[slot],
                    device_id=(s,),
                    device_id_type=pl.DeviceIdType.MESH,
                )
                recv.wait_recv()
            @pl.when(my == s)
            def _(s=s):
                for j in range(N_DEV):
                    if j == s:
                        continue
                    drain = pltpu.make_async_remote_copy(
                        src_ref=send_buf.at[j],
                        dst_ref=kv_buf.at[0],
                        send_sem=a2a_send_sems.at[j],
                        recv_sem=a2a_recv_sems.at[0],
                        device_id=(j,),
                        device_id_type=pl.DeviceIdType.MESH,
                    )
                    drain.wait_send()

        ctx_heads = []
        for h in range(HQ_LOC):
            q_h = q[:, :, 64 * h:64 * h + 64]
            s_chunks = []
            for c in range(len(SENDERS)):
                k_ch = kv_buf[c, 0][:, :, h, :]
                s_chunks.append(lax.dot_general(
                    q_h, k_ch, (((2,), (2,)), ((0,), (0,))),
                    preferred_element_type=jnp.float32,
                ))
            s = jnp.concatenate(s_chunks, axis=2) * 0.125
            s = jnp.where(mask[None], s, -1e9)
            m = jnp.max(s, axis=-1, keepdims=True)
            w = jnp.exp(s - m)
            w = (w / jnp.sum(w, axis=-1, keepdims=True)).astype(jnp.bfloat16)
            c_h = jnp.zeros((B, SQ, DH), jnp.float32)
            for c in range(len(SENDERS)):
                v_ch = kv_buf[c, 1][:, :, h, :]
                c_h = c_h + lax.dot_general(
                    w[:, :, 128 * c:128 * c + 128], v_ch,
                    (((2,), (1,)), ((0,), (0,))),
                    preferred_element_type=jnp.float32,
                )
            ctx_heads.append(c_h)
        ctx = jnp.concatenate(ctx_heads, axis=2).astype(jnp.bfloat16)

        partial = lax.dot_general(
            ctx, wo_ref[...].astype(jnp.bfloat16), (((2,), (0,)), ((), ())),
            preferred_element_type=jnp.float32,
        )

        acc_buf[pl.ds(my, 1)] = partial.astype(jnp.bfloat16)[None]
        bcasts = []
        for d in range(1, N_DEV):
            tgt = (my + d) % N_DEV
            rdma = pltpu.make_async_remote_copy(
                src_ref=acc_buf.at[my],
                dst_ref=acc_buf.at[my],
                send_sem=bc_send_sems.at[tgt],
                recv_sem=bc_recv_sems.at[my],
                device_id=(tgt,),
                device_id_type=pl.DeviceIdType.MESH,
            )
            rdma.start()
            bcasts.append(rdma)
        for d in range(1, N_DEV):
            src = (my + d) % N_DEV
            recv = pltpu.make_async_remote_copy(
                src_ref=acc_buf.at[0],
                dst_ref=acc_buf.at[src],
                send_sem=bc_send_sems.at[0],
                recv_sem=bc_recv_sems.at[src],
                device_id=(src,),
                device_id_type=pl.DeviceIdType.MESH,
            )
            recv.wait_recv()
        for rdma in bcasts:
            rdma.wait_send()

        out_ref[...] = jnp.sum(acc_buf[...].astype(jnp.float32), axis=0)

        def _exit(second_barrier):
            for d in range(1, N_DEV):
                pl.semaphore_signal(
                    second_barrier, inc=1,
                    device_id=((my + d) % N_DEV,),
                    device_id_type=pl.DeviceIdType.MESH,
                )
            pl.semaphore_wait(second_barrier, N_DEV - 1)
        pl.run_scoped(_exit, second_barrier=pltpu.SemaphoreType.REGULAR)

    return pl.pallas_call(
        body,
        out_shape=jax.ShapeDtypeStruct((B, SQ, D_MODEL), jnp.float32),
        in_specs=[pl.BlockSpec(memory_space=pltpu.VMEM)] * 5,
        out_specs=pl.BlockSpec(memory_space=pltpu.VMEM),
        scratch_shapes=[
            pltpu.VMEM((N_DEV, 2, B, SKV_LOC, HQ_LOC, DH), jnp.bfloat16),
            pltpu.VMEM((len(SENDERS), 2, B, SKV_LOC, HQ_LOC, DH), jnp.bfloat16),
            pltpu.VMEM((N_DEV, B, SQ, D_MODEL), jnp.bfloat16),
            pltpu.SemaphoreType.DMA((N_DEV,)),
            pltpu.SemaphoreType.DMA((len(SENDERS),)),
            pltpu.SemaphoreType.DMA((N_DEV,)),
            pltpu.SemaphoreType.DMA((N_DEV,)),
        ],
        compiler_params=pltpu.CompilerParams(collective_id=0),
    )(x, Wq, K_ext, V_ext, Wo)


# device time: 22324 ns/iter; 1.8004x vs baseline; 1.4819x over previous
import jax
import jax.numpy as jnp
from jax import lax
from jax.experimental import pallas as pl
from jax.experimental.pallas import tpu as pltpu

N_DEV = 4
B, SQ, SKV_LOC, HQ_LOC, DH = 2, 128, 128, 4, 64
D_MODEL = 512
HD_LOC = HQ_LOC * DH
SENDERS = (0, 2)
N_CH = len(SENDERS)
HALF = 64


def kernel(x, Wq, K_ext, V_ext, Wo):
    K2 = K_ext.reshape(B, SKV_LOC, N_DEV * HD_LOC)
    V2 = V_ext.reshape(B, SKV_LOC, N_DEV * HD_LOC)

    def body(x_ref, wq_ref, k_ref, v_ref, wo_ref, out_ref,
             send_buf, kv_buf, bf_send, bf_recv,
             a2a_send_sems, a2a_recv_sems, bf_send_sems, bf_recv_sems):
        my = lax.axis_index("i")

        PEERS = {0: (1, 2, 3), 1: (0, 2), 2: (0, 1, 3), 3: (0, 2)}

        def partner_barrier(sem):
            for i in range(N_DEV):
                @pl.when(my == i)
                def _(i=i):
                    for pr in PEERS[i]:
                        pl.semaphore_signal(
                            sem, inc=1,
                            device_id=(pr,),
                            device_id_type=pl.DeviceIdType.MESH,
                        )
                    pl.semaphore_wait(sem, len(PEERS[i]))

        partner_barrier(pltpu.get_barrier_semaphore())

        for slot, s in enumerate(SENDERS):
            @pl.when(my == s)
            def _(slot=slot, s=s):
                dsts = sorted((j for j in range(N_DEV) if j != s),
                              key=lambda j: 0 if j == (s + 2) % N_DEV else 1)
                for p in range(2):
                    for t, t_ref in ((0, k_ref), (1, v_ref)):
                        for j in dsts:
                            send_buf[j, p, t] = t_ref[
                                :, HALF * p:HALF * (p + 1),
                                HD_LOC * j:HD_LOC * (j + 1)].astype(jnp.bfloat16)
                            rdma = pltpu.make_async_remote_copy(
                                src_ref=send_buf.at[j, p, t],
                                dst_ref=kv_buf.at[p, t, :,
                                                  pl.ds(HALF * slot, HALF), :],
                                send_sem=a2a_send_sems.at[j, p, t],
                                recv_sem=a2a_recv_sems.at[slot, p, t],
                                device_id=(j,),
                                device_id_type=pl.DeviceIdType.MESH,
                            )
                            rdma.start()
                for p in range(2):
                    kv_buf[p, 0, :, HALF * slot:HALF * (slot + 1), :] = \
                        k_ref[:, HALF * p:HALF * (p + 1),
                              HD_LOC * s:HD_LOC * (s + 1)].astype(jnp.bfloat16)
                    kv_buf[p, 1, :, HALF * slot:HALF * (slot + 1), :] = \
                        v_ref[:, HALF * p:HALF * (p + 1),
                              HD_LOC * s:HD_LOC * (s + 1)].astype(jnp.bfloat16)

        x_b = x_ref[...].astype(jnp.bfloat16)
        wq_b = wq_ref[...].astype(jnp.bfloat16)
        q = lax.dot_general(
            x_b, wq_b, (((2,), (0,)), ((), ())),
            preferred_element_type=jnp.float32,
        ).astype(jnp.bfloat16)

        def wait_half(p, t):
            for slot, s in enumerate(SENDERS):
                @pl.when(my != s)
                def _(slot=slot, s=s):
                    recv = pltpu.make_async_remote_copy(
                        src_ref=send_buf.at[0, 0, 0],
                        dst_ref=kv_buf.at[p, t, :,
                                          pl.ds(HALF * slot, HALF), :],
                        send_sem=a2a_send_sems.at[0, 0, 0],
                        recv_sem=a2a_recv_sems.at[slot, p, t],
                        device_id=(s,),
                        device_id_type=pl.DeviceIdType.MESH,
                    )
                    recv.wait_recv()

        def softmax_half(p):
            q_p = q[:, HALF * p:HALF * (p + 1), :]
            k_p = kv_buf[p, 0]
            ws = []
            for h in range(HQ_LOC):
                s = lax.dot_general(
                    q_p[:, :, DH * h:DH * (h + 1)], k_p[:, :, DH * h:DH * (h + 1)],
                    (((2,), (2,)), ((0,), (0,))),
                    preferred_element_type=jnp.float32,
                ) * 0.125
                m = jnp.max(s, axis=-1, keepdims=True)
                w = jnp.exp(s - m)
                ws.append((w / jnp.sum(w, axis=-1, keepdims=True)).astype(jnp.bfloat16))
            return ws

        def project_half(p, ws):
            v_p = kv_buf[p, 1]
            ctx = jnp.concatenate([lax.dot_general(
                ws[h], v_p[:, :, DH * h:DH * (h + 1)],
                (((2,), (1,)), ((0,), (0,))),
                preferred_element_type=jnp.float32,
            ) for h in range(HQ_LOC)], axis=2).astype(jnp.bfloat16)
            return lax.dot_general(
                ctx, wo_ref[...].astype(jnp.bfloat16), (((2,), (0,)), ((), ())),
                preferred_element_type=jnp.float32,
            )

        partners = [my ^ 1, 3 - my]

        def ar_start(p, st, acc16):
            bf_send[p, st] = acc16
            rdma = pltpu.make_async_remote_copy(
                src_ref=bf_send.at[p, st],
                dst_ref=bf_recv.at[p, st],
                send_sem=bf_send_sems.at[p, st],
                recv_sem=bf_recv_sems.at[p, st],
                device_id=(partners[st],),
                device_id_type=pl.DeviceIdType.MESH,
            )
            rdma.start()
            return rdma

        wait_half(0, 0)
        w0 = softmax_half(0)
        wait_half(0, 1)
        a0 = project_half(0, w0).astype(jnp.bfloat16)
        d00 = ar_start(0, 0, a0)

        wait_half(1, 0)
        w1 = softmax_half(1)
        wait_half(1, 1)
        a1 = project_half(1, w1).astype(jnp.bfloat16)
        d10 = ar_start(1, 0, a1)

        d00.wait_recv()
        a0 = a0 + bf_recv[0, 0]
        d01 = ar_start(0, 1, a0)

        d10.wait_recv()
        a1 = a1 + bf_recv[1, 0]
        d11 = ar_start(1, 1, a1)

        d01.wait_recv()
        out_ref[:, 0:HALF, :] = a0 + bf_recv[0, 1]
        d11.wait_recv()
        out_ref[:, HALF:SQ, :] = a1 + bf_recv[1, 1]

        for d in (d00, d10, d01, d11):
            d.wait_send()

        for s in SENDERS:
            @pl.when(my == s)
            def _(s=s):
                for p in range(2):
                    for t in range(2):
                        for j in range(N_DEV):
                            if j == s:
                                continue
                            drain = pltpu.make_async_remote_copy(
                                src_ref=send_buf.at[j, p, t],
                                dst_ref=kv_buf.at[0, 0, :,
                                                  pl.ds(0, HALF), :],
                                send_sem=a2a_send_sems.at[j, p, t],
                                recv_sem=a2a_recv_sems.at[0, 0, 0],
                                device_id=(j,),
                                device_id_type=pl.DeviceIdType.MESH,
                            )
                            drain.wait_send()

        pl.run_scoped(partner_barrier,
                      sem=pltpu.SemaphoreType.REGULAR)

    return pl.pallas_call(
        body,
        out_shape=jax.ShapeDtypeStruct((B, SQ, D_MODEL), jnp.bfloat16),
        in_specs=[pl.BlockSpec(memory_space=pltpu.VMEM)] * 5,
        out_specs=pl.BlockSpec(memory_space=pltpu.VMEM),
        scratch_shapes=[
            pltpu.VMEM((N_DEV, 2, 2, B, HALF, HD_LOC), jnp.bfloat16),
            pltpu.VMEM((2, 2, B, N_CH * HALF, HD_LOC), jnp.bfloat16),
            pltpu.VMEM((2, 2, B, HALF, D_MODEL), jnp.bfloat16),
            pltpu.VMEM((2, 2, B, HALF, D_MODEL), jnp.bfloat16),
            pltpu.SemaphoreType.DMA((N_DEV, 2, 2)),
            pltpu.SemaphoreType.DMA((N_CH, 2, 2)),
            pltpu.SemaphoreType.DMA((2, 2)),
            pltpu.SemaphoreType.DMA((2, 2)),
        ],
        compiler_params=pltpu.CompilerParams(collective_id=0),
    )(x, Wq, K2, V2, Wo)
